# baseline (speedup 1.0000x reference)
import dataclasses

import jax
import jax.numpy as jnp
from jax import lax
from jax.experimental import pallas as pl
from jax.experimental.pallas import tpu as pltpu
from jax.experimental.pallas import tpu_sc as plsc

B = 16384
D = 64
C_POS = 20
C_NEG = 50
C = C_POS + C_NEG
C_PAD = 128
NW = 32
BPW = B // NW
NB = 8
STEPS = BPW // NB
ROWS_STEP = NB * C
GCHUNK = 112
NGC = ROWS_STEP // GCHUNK
VPAIR = 500000


def _sc_body(u_hbm, v_hbm, idx_hbm, x_hbm, out_hbm,
             xbuf, xp, idx_v, idxp_v, vc_step, rows, out_v, sem):
    wid = lax.axis_index("s") * 2 + lax.axis_index("c")
    base = wid * BPW

    lane = lax.iota(jnp.int32, 16)
    masks = [lane == j for j in range(16)]

    pltpu.sync_copy(x_hbm.at[pl.ds(base, BPW)], xbuf.at[pl.ds(0, BPW)])
    for k in range(BPW // 16):
        xv = xbuf[pl.ds(k * 16, 16)]
        xp[pl.ds(k * 16, 16)] = lax.shift_right_logical(xv, 1)

    @pl.loop(0, STEPS)
    def _step(s):
        b0 = base + s * NB
        pltpu.sync_copy(idx_hbm.at[pl.ds(b0 * C, ROWS_STEP)],
                        idx_v.at[pl.ds(0, ROWS_STEP)])
        for k in range(ROWS_STEP // 16):
            iv = idx_v[pl.ds(k * 16, 16)]
            idxp_v[pl.ds(k * 16, 16)] = lax.shift_right_logical(iv, 1)
        copies = [
            pltpu.async_copy(
                u_hbm.at[idxp_v.at[pl.ds(k * GCHUNK, GCHUNK)]],
                rows.at[pl.ds(k * GCHUNK, GCHUNK)], sem)
            for k in range(NGC)
        ]
        copies.append(
            pltpu.async_copy(
                v_hbm.at[xp.at[pl.ds(s * NB, NB)]], vc_step, sem))
        for cp in copies:
            cp.wait()

        @pl.loop(0, NB)
        def _row(i):
            bb = s * NB + i
            vbase = (xbuf[pl.ds(bb, 16)][0] & 1) * D
            vc0 = vc_step[i, pl.ds(vbase, 16)]
            vc1 = vc_step[i, pl.ds(vbase + 16, 16)]
            vc2 = vc_step[i, pl.ds(vbase + 32, 16)]
            vc3 = vc_step[i, pl.ds(vbase + 48, 16)]
            accs = [jnp.zeros((16,), jnp.float32) for _ in range(5)]
            for j in range(C):
                r = i * C + j
                ubase = (idx_v[pl.ds(r, 16)][0] & 1) * D
                t = rows[r, pl.ds(ubase, 16)] * vc0
                t = t + rows[r, pl.ds(ubase + 16, 16)] * vc1
                t = t + rows[r, pl.ds(ubase + 32, 16)] * vc2
                t = t + rows[r, pl.ds(ubase + 48, 16)] * vc3
                sv = jnp.sum(t)
                g, l = divmod(j, 16)
                accs[g] = jnp.where(masks[l], sv, accs[g])
            for g in range(5):
                out_v[i, pl.ds(g * 16, 16)] = accs[g]
            out_v[i, pl.ds(80, 16)] = accs[4] * 0.0
            out_v[i, pl.ds(96, 16)] = accs[4] * 0.0
            out_v[i, pl.ds(112, 16)] = accs[4] * 0.0

        pltpu.sync_copy(out_v, out_hbm.at[pl.ds(b0, NB)])


@jax.jit
def _sc_dots(u_weight, v_weight, idx_all, x):
    mesh = plsc.VectorSubcoreMesh(core_axis_name="c", subcore_axis_name="s")
    cp = pltpu.CompilerParams()
    if "needs_layout_passes" in pltpu.CompilerParams.__dataclass_fields__:
        cp = dataclasses.replace(cp, needs_layout_passes=False)
    kern = pl.kernel(
        _sc_body,
        out_type=jax.ShapeDtypeStruct((B, C_PAD), jnp.float32),
        mesh=mesh,
        scratch_types=[
            pltpu.VMEM((BPW + 16,), jnp.int32),
            pltpu.VMEM((BPW,), jnp.int32),
            pltpu.VMEM((ROWS_STEP + 16,), jnp.int32),
            pltpu.VMEM((ROWS_STEP,), jnp.int32),
            pltpu.VMEM((NB, 128), jnp.float32),
            pltpu.VMEM((ROWS_STEP, 128), jnp.float32),
            pltpu.VMEM((NB, C_PAD), jnp.float32),
            pltpu.SemaphoreType.DMA,
        ],
        compiler_params=cp,
    )
    return kern(u_weight, v_weight, idx_all, x)


def _tc_body(uv_ref, o_ref):
    z = uv_ref[...]
    col = lax.broadcasted_iota(jnp.int32, z.shape, 1)
    pos = col < C_POS
    valid = col < C
    zs = jnp.where(pos, z, -z)
    ls = jnp.minimum(zs, 0.0) - jnp.log1p(jnp.exp(-jnp.abs(zs)))
    contrib = jnp.where(valid, ls, 0.0)
    o_ref[...] = -jnp.sum(contrib, axis=1)


@jax.jit
def _tc_epilogue(uv):
    blk = 2048
    return pl.pallas_call(
        _tc_body,
        grid=(B // blk,),
        in_specs=[pl.BlockSpec((blk, C_PAD), lambda i: (i, 0))],
        out_specs=pl.BlockSpec((blk,), lambda i: (i,)),
        out_shape=jax.ShapeDtypeStruct((B,), jnp.float32),
    )(uv)


def kernel(x, positive_w, negative_w, V_weight, U_weight):
    idx_all = jnp.concatenate(
        [positive_w.astype(jnp.int32), negative_w.astype(jnp.int32)], axis=1
    ).reshape(-1)
    u2 = U_weight.reshape(VPAIR, 2 * D)
    v2 = V_weight.reshape(VPAIR, 2 * D)
    uv = _sc_dots(u2, v2, idx_all, x.astype(jnp.int32))
    return _tc_epilogue(uv)
